# initial kernel scaffold (unmeasured)
import jax
import jax.numpy as jnp
from jax import lax
from jax.experimental import pallas as pl
from jax.experimental.pallas import tpu as pltpu

N_DEV = 4
B = 4
SQ = 256
SKV = 4096
HQ = 32
HPER = 8
DH = 128
DM = 1024
QB = 64
NQB = SQ // QB
NT = SKV // (4 * QB)
SCALE = 0.08838834764831843

BF = jnp.bfloat16
F32 = jnp.float32


def kernel(x, Wq, K_ext, V_ext, Wo):
    Kr = K_ext.reshape(B, NT, NQB, QB, HQ, DH)
    Vr = V_ext.reshape(B, NT, NQB, QB, HQ, DH)

    def body(x_ref, wq_ref, k_hbm, v_hbm, wo_ref, out_ref,
             comm_x, wq_bf, wo_bf, q_buf, k_tile, v_tile, ctx_buf,
             accum, rs_buf, snd,
             ag_send, ag_recv, rs_send, rs_recv, kv_sem):
        my = lax.axis_index("i")
        left = lax.rem(my + N_DEV - 1, N_DEV)
        right = lax.rem(my + 1, N_DEV)
        h0 = my * HPER

        barrier = pltpu.get_barrier_semaphore()
        for nbr in (left, right):
            pl.semaphore_signal(barrier, inc=1, device_id=(nbr,),
                                device_id_type=pl.DeviceIdType.MESH)
        pl.semaphore_wait(barrier, 2)

        wq_bf[...] = wq_ref[...].astype(BF)
        wo_bf[...] = wo_ref[...].astype(BF)
        comm_x[0] = x_ref[0].astype(BF)

        for h in range(N_DEV - 1):
            rdma = pltpu.make_async_remote_copy(
                src_ref=comm_x.at[h],
                dst_ref=comm_x.at[h + 1],
                send_sem=ag_send.at[h],
                recv_sem=ag_recv.at[h],
                device_id=(right,),
                device_id_type=pl.DeviceIdType.MESH,
            )
            rdma.start()
            rdma.wait()

        for s in range(N_DEV):
            b = lax.rem(my + N_DEV - s, N_DEV)
            q_buf[...] = jnp.dot(comm_x[s], wq_bf[...],
                                 preferred_element_type=F32)
            for qb in range(NQB):
                copies = []
                for h in range(HPER):
                    ck = pltpu.make_async_copy(
                        k_hbm.at[b, :, qb, :, h0 + h, :],
                        k_tile.at[h], kv_sem.at[h])
                    cv = pltpu.make_async_copy(
                        v_hbm.at[b, :, qb, :, h0 + h, :],
                        v_tile.at[h], kv_sem.at[HPER + h])
                    ck.start()
                    cv.start()
                    copies += [ck, cv]
                for c in copies:
                    c.wait()
                for h in range(HPER):
                    qh = q_buf[qb * QB:(qb + 1) * QB,
                               h * DH:(h + 1) * DH].astype(BF)
                    kh = k_tile[h].reshape(NT * QB, DH).astype(BF)
                    sc = lax.dot_general(
                        qh, kh, (((1,), (1,)), ((), ())),
                        preferred_element_type=F32) * SCALE
                    m = jnp.max(sc, axis=1, keepdims=True)
                    w = jnp.exp(sc - m)
                    w = w / jnp.sum(w, axis=1, keepdims=True)
                    vh = v_tile[h].reshape(NT * QB, DH).astype(BF)
                    c = jnp.dot(w.astype(BF), vh,
                                preferred_element_type=F32)
                    ctx_buf[qb * QB:(qb + 1) * QB,
                            h * DH:(h + 1) * DH] = c.astype(BF)
            accum[s] = jnp.dot(ctx_buf[...], wo_bf[...],
                               preferred_element_type=F32)

        snd[0] = accum[1].astype(BF)
        for s in range(N_DEV - 1):
            rdma = pltpu.make_async_remote_copy(
                src_ref=snd.at[s],
                dst_ref=rs_buf.at[s],
                send_sem=rs_send.at[s],
                recv_sem=rs_recv.at[s],
                device_id=(right,),
                device_id_type=pl.DeviceIdType.MESH,
            )
            rdma.start()
            rdma.wait()
            if s < N_DEV - 2:
                snd[s + 1] = (rs_buf[s].astype(F32)
                              + accum[s + 2]).astype(BF)
        out_ref[0] = rs_buf[N_DEV - 2].astype(F32) + accum[0]

    return pl.pallas_call(
        body,
        out_shape=jax.ShapeDtypeStruct((1, SQ, DM), jnp.float32),
        in_specs=[
            pl.BlockSpec(memory_space=pltpu.VMEM),
            pl.BlockSpec(memory_space=pltpu.VMEM),
            pl.BlockSpec(memory_space=pltpu.ANY),
            pl.BlockSpec(memory_space=pltpu.ANY),
            pl.BlockSpec(memory_space=pltpu.VMEM),
        ],
        out_specs=pl.BlockSpec(memory_space=pltpu.VMEM),
        scratch_shapes=[
            pltpu.VMEM((N_DEV, SQ, DM), BF),
            pltpu.VMEM((DM, DM), BF),
            pltpu.VMEM((DM, DM), BF),
            pltpu.VMEM((SQ, DM), F32),
            pltpu.VMEM((HPER, NT, QB, DH), F32),
            pltpu.VMEM((HPER, NT, QB, DH), F32),
            pltpu.VMEM((SQ, DM), BF),
            pltpu.VMEM((N_DEV, SQ, DM), F32),
            pltpu.VMEM((N_DEV - 1, SQ, DM), BF),
            pltpu.VMEM((N_DEV - 1, SQ, DM), BF),
            pltpu.SemaphoreType.DMA((N_DEV - 1,)),
            pltpu.SemaphoreType.DMA((N_DEV - 1,)),
            pltpu.SemaphoreType.DMA((N_DEV - 1,)),
            pltpu.SemaphoreType.DMA((N_DEV - 1,)),
            pltpu.SemaphoreType.DMA((2 * HPER,)),
        ],
        compiler_params=pltpu.CompilerParams(collective_id=0),
    )(x, Wq, Kr, Vr, Wo)


# baseline (device time: 177033 ns/iter reference)
import jax
import jax.numpy as jnp
from jax import lax
from jax.experimental import pallas as pl
from jax.experimental.pallas import tpu as pltpu

N_DEV = 4
B = 4
SQ = 256
SKV = 4096
HQ = 32
HPER = 8
DH = 128
DM = 1024
QB = 64
NQB = SQ // QB
NT = SKV // (4 * QB)
SCALE = 0.08838834764831843

BF = jnp.bfloat16
F32 = jnp.float32


def kernel(x, Wq, K_ext, V_ext, Wo):
    Kr = K_ext.reshape(B, NT, NQB, QB, HQ, DH)
    Vr = V_ext.reshape(B, NT, NQB, QB, HQ, DH)

    def body(x_ref, wq_ref, k_hbm, v_hbm, wo_ref, out_ref,
             comm_x, wq_bf, wo_bf, q_buf, k_tile, v_tile, ctx_buf,
             accum, rs_buf, snd,
             ag_send, ag_recv, rs_send, rs_recv, kv_sem):
        my = lax.axis_index("i")
        left = lax.rem(my + N_DEV - 1, N_DEV)
        right = lax.rem(my + 1, N_DEV)
        h0 = my * HPER

        barrier = pltpu.get_barrier_semaphore()
        for nbr in (left, right):
            pl.semaphore_signal(barrier, inc=1, device_id=(nbr,),
                                device_id_type=pl.DeviceIdType.MESH)
        pl.semaphore_wait(barrier, 2)

        wq_bf[...] = wq_ref[...].astype(BF)
        wo_bf[...] = wo_ref[...].astype(BF)
        comm_x[0] = x_ref[0].astype(BF)

        for h in range(N_DEV - 1):
            rdma = pltpu.make_async_remote_copy(
                src_ref=comm_x.at[h],
                dst_ref=comm_x.at[h + 1],
                send_sem=ag_send.at[h],
                recv_sem=ag_recv.at[h],
                device_id=(right,),
                device_id_type=pl.DeviceIdType.MESH,
            )
            rdma.start()
            rdma.wait()

        for s in range(N_DEV):
            b = lax.rem(my + N_DEV - s, N_DEV)
            q_buf[...] = jnp.dot(comm_x[s], wq_bf[...],
                                 preferred_element_type=F32)
            for qb in range(NQB):
                copies = []
                for h in range(HPER):
                    ck = pltpu.make_async_copy(
                        k_hbm.at[b, :, qb, :, h0 + h, :],
                        k_tile.at[h], kv_sem.at[h])
                    cv = pltpu.make_async_copy(
                        v_hbm.at[b, :, qb, :, h0 + h, :],
                        v_tile.at[h], kv_sem.at[HPER + h])
                    ck.start()
                    cv.start()
                    copies += [ck, cv]
                for c in copies:
                    c.wait()
                for h in range(HPER):
                    qh = q_buf[qb * QB:(qb + 1) * QB,
                               h * DH:(h + 1) * DH].astype(BF)
                    kh = k_tile[h].reshape(NT * QB, DH).astype(BF)
                    sc = lax.dot_general(
                        qh, kh, (((1,), (1,)), ((), ())),
                        preferred_element_type=F32) * SCALE
                    m = jnp.max(sc, axis=1, keepdims=True)
                    w = jnp.exp(sc - m)
                    w = w / jnp.sum(w, axis=1, keepdims=True)
                    vh = v_tile[h].reshape(NT * QB, DH).astype(BF)
                    c = jnp.dot(w.astype(BF), vh,
                                preferred_element_type=F32)
                    ctx_buf[qb * QB:(qb + 1) * QB,
                            h * DH:(h + 1) * DH] = c.astype(BF)
            accum[s] = jnp.dot(ctx_buf[...], wo_bf[...],
                               preferred_element_type=F32)

        snd[0] = accum[1].astype(BF)
        for s in range(N_DEV - 1):
            rdma = pltpu.make_async_remote_copy(
                src_ref=snd.at[s],
                dst_ref=rs_buf.at[s],
                send_sem=rs_send.at[s],
                recv_sem=rs_recv.at[s],
                device_id=(right,),
                device_id_type=pl.DeviceIdType.MESH,
            )
            rdma.start()
            rdma.wait()
            if s < N_DEV - 2:
                snd[s + 1] = (rs_buf[s].astype(F32)
                              + accum[s + 2]).astype(BF)
        out_ref[0] = rs_buf[N_DEV - 2].astype(F32) + accum[0]

    return pl.pallas_call(
        body,
        out_shape=jax.ShapeDtypeStruct((1, SQ, DM), jnp.float32),
        in_specs=[
            pl.BlockSpec(memory_space=pltpu.VMEM),
            pl.BlockSpec(memory_space=pltpu.VMEM),
            pl.BlockSpec(memory_space=pl.ANY),
            pl.BlockSpec(memory_space=pl.ANY),
            pl.BlockSpec(memory_space=pltpu.VMEM),
        ],
        out_specs=pl.BlockSpec(memory_space=pltpu.VMEM),
        scratch_shapes=[
            pltpu.VMEM((N_DEV, SQ, DM), BF),
            pltpu.VMEM((DM, DM), BF),
            pltpu.VMEM((DM, DM), BF),
            pltpu.VMEM((SQ, DM), F32),
            pltpu.VMEM((HPER, NT, QB, DH), F32),
            pltpu.VMEM((HPER, NT, QB, DH), F32),
            pltpu.VMEM((SQ, DM), BF),
            pltpu.VMEM((N_DEV, SQ, DM), F32),
            pltpu.VMEM((N_DEV - 1, SQ, DM), BF),
            pltpu.VMEM((N_DEV - 1, SQ, DM), BF),
            pltpu.SemaphoreType.DMA((N_DEV - 1,)),
            pltpu.SemaphoreType.DMA((N_DEV - 1,)),
            pltpu.SemaphoreType.DMA((N_DEV - 1,)),
            pltpu.SemaphoreType.DMA((N_DEV - 1,)),
            pltpu.SemaphoreType.DMA((2 * HPER,)),
        ],
        compiler_params=pltpu.CompilerParams(collective_id=0),
    )(x, Wq, Kr, Vr, Wo)


# device time: 78806 ns/iter; 2.2464x vs baseline; 2.2464x over previous
import jax
import jax.numpy as jnp
from jax import lax
from jax.experimental import pallas as pl
from jax.experimental.pallas import tpu as pltpu

N_DEV = 4
B = 4
SQ = 256
SKV = 4096
HQ = 32
HPER = 8
DH = 128
DM = 1024
QB = 64
NQB = SQ // QB
NT = SKV // (4 * QB)
NTILES = N_DEV * NQB
SCALE = 0.08838834764831843

BF = jnp.bfloat16
F32 = jnp.float32


def kernel(x, Wq, K_ext, V_ext, Wo):
    Kr = K_ext.reshape(B, NT, NQB, QB, HQ, DH)
    Vr = V_ext.reshape(B, NT, NQB, QB, HQ, DH)

    def body(x_ref, wq_ref, k_hbm, v_hbm, wo_ref, out_ref,
             comm_x, wq_bf, wo_bf, q_buf, k_tile, v_tile, ctx_buf,
             accum, rs_buf, snd,
             ag_send, ag_recv, rs_send, rs_recv, kv_sem):
        my = lax.axis_index("i")
        left = lax.rem(my + N_DEV - 1, N_DEV)
        right = lax.rem(my + 1, N_DEV)
        h0 = my * HPER

        barrier = pltpu.get_barrier_semaphore()
        for nbr in (left, right):
            pl.semaphore_signal(barrier, inc=1, device_id=(nbr,),
                                device_id_type=pl.DeviceIdType.MESH)
        pl.semaphore_wait(barrier, 2)

        def ag(h):
            return pltpu.make_async_remote_copy(
                src_ref=comm_x.at[h],
                dst_ref=comm_x.at[h + 1],
                send_sem=ag_send.at[h],
                recv_sem=ag_recv.at[h],
                device_id=(right,),
                device_id_type=pl.DeviceIdType.MESH,
            )

        def rs(s):
            return pltpu.make_async_remote_copy(
                src_ref=snd.at[s],
                dst_ref=rs_buf.at[s],
                send_sem=rs_send.at[s],
                recv_sem=rs_recv.at[s],
                device_id=(right,),
                device_id_type=pl.DeviceIdType.MESH,
            )

        comm_x[0] = x_ref[0].astype(BF)
        ag(0).start()

        wq_bf[...] = (wq_ref[...] * SCALE).astype(BF)
        wo_bf[...] = wo_ref[...].astype(BF)

        bs = [lax.rem(my + N_DEV - s, N_DEV) for s in range(N_DEV)]

        def tile_copies(i, buf):
            s, qb = divmod(i, NQB)
            b = bs[s]
            cps = []
            for h in range(HPER):
                cps.append(pltpu.make_async_copy(
                    k_hbm.at[b, :, qb, :, h0 + h, :],
                    k_tile.at[buf, h], kv_sem.at[buf, h]))
                cps.append(pltpu.make_async_copy(
                    v_hbm.at[b, :, qb, :, h0 + h, :],
                    v_tile.at[buf, h], kv_sem.at[buf, HPER + h]))
            return cps

        for c in tile_copies(0, 0):
            c.start()

        for i in range(NTILES):
            s, qb = divmod(i, NQB)
            buf = i % 2
            if qb == 0:
                if s > 0:
                    ag(s - 1).wait()
                    if s < N_DEV - 1:
                        ag(s).start()
                q_buf[...] = jnp.dot(comm_x[s], wq_bf[...],
                                     preferred_element_type=F32).astype(BF)
            if i + 1 < NTILES:
                for c in tile_copies(i + 1, (i + 1) % 2):
                    c.start()
            for c in tile_copies(i, buf):
                c.wait()
            for h in range(HPER):
                qh = q_buf[qb * QB:(qb + 1) * QB, h * DH:(h + 1) * DH]
                kh = k_tile[buf, h].reshape(NT * QB, DH).astype(BF)
                sc = lax.dot_general(qh, kh, (((1,), (1,)), ((), ())),
                                     preferred_element_type=F32)
                m = jnp.max(sc, axis=1, keepdims=True)
                e = jnp.exp(sc - m)
                ssum = jnp.sum(e, axis=1, keepdims=True)
                vh = v_tile[buf, h].reshape(NT * QB, DH).astype(BF)
                c = jnp.dot(e.astype(BF), vh,
                            preferred_element_type=F32) / ssum
                ctx_buf[qb * QB:(qb + 1) * QB,
                        h * DH:(h + 1) * DH] = c.astype(BF)
            if qb == NQB - 1:
                accum[s] = jnp.dot(ctx_buf[...], wo_bf[...],
                                   preferred_element_type=F32)
                if s == 1:
                    snd[0] = accum[1].astype(BF)
                    rs(0).start()
                elif s == 2:
                    rs(0).wait()
                    snd[1] = (rs_buf[0].astype(F32)
                              + accum[2]).astype(BF)
                    rs(1).start()
                elif s == 3:
                    rs(1).wait()
                    snd[2] = (rs_buf[1].astype(F32)
                              + accum[3]).astype(BF)
                    last = rs(2)
                    last.start()
                    last.wait()
                    out_ref[0] = rs_buf[2].astype(F32) + accum[0]

    return pl.pallas_call(
        body,
        out_shape=jax.ShapeDtypeStruct((1, SQ, DM), jnp.float32),
        in_specs=[
            pl.BlockSpec(memory_space=pltpu.VMEM),
            pl.BlockSpec(memory_space=pltpu.VMEM),
            pl.BlockSpec(memory_space=pl.ANY),
            pl.BlockSpec(memory_space=pl.ANY),
            pl.BlockSpec(memory_space=pltpu.VMEM),
        ],
        out_specs=pl.BlockSpec(memory_space=pltpu.VMEM),
        scratch_shapes=[
            pltpu.VMEM((N_DEV, SQ, DM), BF),
            pltpu.VMEM((DM, DM), BF),
            pltpu.VMEM((DM, DM), BF),
            pltpu.VMEM((SQ, DM), BF),
            pltpu.VMEM((2, HPER, NT, QB, DH), F32),
            pltpu.VMEM((2, HPER, NT, QB, DH), F32),
            pltpu.VMEM((SQ, DM), BF),
            pltpu.VMEM((N_DEV, SQ, DM), F32),
            pltpu.VMEM((N_DEV - 1, SQ, DM), BF),
            pltpu.VMEM((N_DEV - 1, SQ, DM), BF),
            pltpu.SemaphoreType.DMA((N_DEV - 1,)),
            pltpu.SemaphoreType.DMA((N_DEV - 1,)),
            pltpu.SemaphoreType.DMA((N_DEV - 1,)),
            pltpu.SemaphoreType.DMA((N_DEV - 1,)),
            pltpu.SemaphoreType.DMA((2, 2 * HPER)),
        ],
        compiler_params=pltpu.CompilerParams(collective_id=0),
    )(x, Wq, Kr, Vr, Wo)


# device time: 67386 ns/iter; 2.6271x vs baseline; 1.1695x over previous
import jax
import jax.numpy as jnp
from jax import lax
from jax.experimental import pallas as pl
from jax.experimental.pallas import tpu as pltpu

N_DEV = 4
B = 4
SQ = 256
SKV = 4096
HQ = 32
HPER = 8
DH = 128
DM = 1024
QB = 64
NQB = SQ // QB
NT = SKV // (4 * QB)
NTILES = N_DEV * NQB
SCALE = 0.08838834764831843

BF = jnp.bfloat16
F32 = jnp.float32


def kernel(x, Wq, K_ext, V_ext, Wo):
    Kr = K_ext.reshape(B, NT, NQB, QB, HQ, DH)
    Vr = V_ext.reshape(B, NT, NQB, QB, HQ, DH)

    def body(x_ref, wq_ref, k_hbm, v_hbm, wo_ref, out_ref,
             comm_x, wq_bf, wo_bf, q_buf, k_tile, v_tile,
             accum0, rs_buf, snd,
             ag_send, ag_recv, rs_send, rs_recv, kv_sem):
        my = lax.axis_index("i")
        left = lax.rem(my + N_DEV - 1, N_DEV)
        right = lax.rem(my + 1, N_DEV)
        h0 = my * HPER

        barrier = pltpu.get_barrier_semaphore()
        for nbr in (left, right):
            pl.semaphore_signal(barrier, inc=1, device_id=(nbr,),
                                device_id_type=pl.DeviceIdType.MESH)
        pl.semaphore_wait(barrier, 2)

        def ag(h):
            return pltpu.make_async_remote_copy(
                src_ref=comm_x.at[h],
                dst_ref=comm_x.at[h + 1],
                send_sem=ag_send.at[h],
                recv_sem=ag_recv.at[h],
                device_id=(right,),
                device_id_type=pl.DeviceIdType.MESH,
            )

        def rs(st, r):
            return pltpu.make_async_remote_copy(
                src_ref=snd.at[st, r],
                dst_ref=rs_buf.at[st, r],
                send_sem=rs_send.at[st, r],
                recv_sem=rs_recv.at[st, r],
                device_id=(right,),
                device_id_type=pl.DeviceIdType.MESH,
            )

        comm_x[0] = x_ref[0].astype(BF)
        ag(0).start()

        wq_bf[...] = (wq_ref[...] * SCALE).astype(BF)
        wo_bf[...] = wo_ref[...].astype(BF)

        bs = [lax.rem(my + N_DEV - s, N_DEV) for s in range(N_DEV)]

        def tile_copies(i, buf):
            s, qb = divmod(i, NQB)
            b = bs[s]
            cps = []
            for h in range(HPER):
                cps.append(pltpu.make_async_copy(
                    k_hbm.at[b, :, qb, :, h0 + h, :],
                    k_tile.at[buf, h], kv_sem.at[buf, h]))
                cps.append(pltpu.make_async_copy(
                    v_hbm.at[b, :, qb, :, h0 + h, :],
                    v_tile.at[buf, h], kv_sem.at[buf, HPER + h]))
            return cps

        for c in tile_copies(0, 0):
            c.start()

        for i in range(NTILES):
            s, qb = divmod(i, NQB)
            buf = i % 2
            if qb == 0:
                if s > 0:
                    ag(s - 1).wait()
                    if s < N_DEV - 1:
                        ag(s).start()
                q_buf[...] = jnp.dot(comm_x[s], wq_bf[...],
                                     preferred_element_type=F32).astype(BF)
            if i + 1 < NTILES:
                for c in tile_copies(i + 1, (i + 1) % 2):
                    c.start()
            for c in tile_copies(i, buf):
                c.wait()
            parts = []
            for h in range(HPER):
                qh = q_buf[qb * QB:(qb + 1) * QB, h * DH:(h + 1) * DH]
                kh = k_tile[buf, h].reshape(NT * QB, DH).astype(BF)
                sc = lax.dot_general(qh, kh, (((1,), (1,)), ((), ())),
                                     preferred_element_type=F32)
                e = jnp.exp(sc)
                ssum = jnp.sum(e, axis=1, keepdims=True)
                vh = v_tile[buf, h].reshape(NT * QB, DH).astype(BF)
                parts.append(jnp.dot(e.astype(BF), vh,
                                     preferred_element_type=F32) / ssum)
            ctx = jnp.concatenate(parts, axis=1).astype(BF)
            block = jnp.dot(ctx, wo_bf[...], preferred_element_type=F32)
            if s == 0:
                accum0[qb] = block
            else:
                st = s - 1
                if st > 0:
                    rs(st - 1, qb).wait()
                    block = block + rs_buf[st - 1, qb].astype(F32)
                snd[st, qb] = block.astype(BF)
                rs(st, qb).start()
        for r in range(NQB):
            rs(N_DEV - 2, r).wait()
            out_ref[0, r * QB:(r + 1) * QB, :] = (
                rs_buf[N_DEV - 2, r].astype(F32) + accum0[r])

    return pl.pallas_call(
        body,
        out_shape=jax.ShapeDtypeStruct((1, SQ, DM), jnp.float32),
        in_specs=[
            pl.BlockSpec(memory_space=pltpu.VMEM),
            pl.BlockSpec(memory_space=pltpu.VMEM),
            pl.BlockSpec(memory_space=pl.ANY),
            pl.BlockSpec(memory_space=pl.ANY),
            pl.BlockSpec(memory_space=pltpu.VMEM),
        ],
        out_specs=pl.BlockSpec(memory_space=pltpu.VMEM),
        scratch_shapes=[
            pltpu.VMEM((N_DEV, SQ, DM), BF),
            pltpu.VMEM((DM, DM), BF),
            pltpu.VMEM((DM, DM), BF),
            pltpu.VMEM((SQ, DM), BF),
            pltpu.VMEM((2, HPER, NT, QB, DH), F32),
            pltpu.VMEM((2, HPER, NT, QB, DH), F32),
            pltpu.VMEM((NQB, QB, DM), F32),
            pltpu.VMEM((N_DEV - 1, NQB, QB, DM), BF),
            pltpu.VMEM((N_DEV - 1, NQB, QB, DM), BF),
            pltpu.SemaphoreType.DMA((N_DEV - 1,)),
            pltpu.SemaphoreType.DMA((N_DEV - 1,)),
            pltpu.SemaphoreType.DMA((N_DEV - 1, NQB)),
            pltpu.SemaphoreType.DMA((N_DEV - 1, NQB)),
            pltpu.SemaphoreType.DMA((2, 2 * HPER)),
        ],
        compiler_params=pltpu.CompilerParams(collective_id=0),
    )(x, Wq, Kr, Vr, Wo)
